# baseline (speedup 1.0000x reference)
import jax
import jax.numpy as jnp
from jax import lax
from jax.experimental import pallas as pl
from jax.experimental.pallas import tpu as pltpu
from jax.experimental.pallas import tpu_sc as plsc

VOCAB = 100000
MAX_POS = 8192
HIDDEN = 1024
BATCH = 4
SEQ = 8192

NUM_ROWS = SEQ * BATCH
NC, NS = 2, 16
NW = NC * NS
ROWS_PER_W = NUM_ROWS // NW
CHUNK = 16
SEQ_PER_CHUNK = CHUNK // BATCH
NCHUNK = ROWS_PER_W // CHUNK
VREGS_PER_ROW = HIDDEN // 16


def _emb_body(widx_hbm, pidx_hbm, word_hbm, pos_hbm, out_hbm,
              widx_v, pidx_v,
              wbuf0, wbuf1, pbuf0, pbuf1, obuf0, obuf1,
              wsem0, wsem1, psem0, psem1, osem0, osem1):
    wbufs = (wbuf0, wbuf1)
    pbufs = (pbuf0, pbuf1)
    obufs = (obuf0, obuf1)
    wsems = (wsem0, wsem1)
    psems = (psem0, psem1)
    osems = (osem0, osem1)

    wid = lax.axis_index("s") * NC + lax.axis_index("c")
    base = wid * ROWS_PER_W

    pltpu.sync_copy(widx_hbm.at[pl.ds(base, ROWS_PER_W)], widx_v)
    pltpu.sync_copy(pidx_hbm.at[pl.ds(base, ROWS_PER_W)], pidx_v)

    def issue_gathers(c, b):
        off = c * CHUNK
        pltpu.async_copy(word_hbm.at[widx_v.at[pl.ds(off, CHUNK)]],
                         wbufs[b], wsems[b])
        pltpu.async_copy(pos_hbm.at[pidx_v.at[pl.ds(off, CHUNK)]],
                         pbufs[b], psems[b])

    def wait_gathers(b):
        pltpu.make_async_copy(word_hbm.at[pl.ds(0, CHUNK)],
                              wbufs[b], wsems[b]).wait()
        pltpu.make_async_copy(pos_hbm.at[pl.ds(0, CHUNK)],
                              pbufs[b], psems[b]).wait()

    def wait_store(b):
        for _ in range(SEQ_PER_CHUNK):
            pltpu.make_async_copy(obufs[b].at[pl.ds(0, BATCH)],
                                  out_hbm.at[0], osems[b]).wait()

    issue_gathers(0, 0)
    issue_gathers(1, 1)

    def step(i, carry):
        for b in range(2):
            c = i * 2 + b
            wait_gathers(b)
            @pl.when(i >= 1)
            def _():
                wait_store(b)
            wb, pb, ob = wbufs[b], pbufs[b], obufs[b]

            def add_row(r, carry2):
                for v in range(VREGS_PER_ROW):
                    sl = pl.ds(v * 16, 16)
                    ob[r, sl] = wb[r, sl] + pb[r, sl]
                return carry2

            lax.fori_loop(0, CHUNK, add_row, 0)

            @pl.when(i < (NCHUNK // 2) - 1)
            def _():
                issue_gathers(c + 2, b)
            s0 = (base + c * CHUNK) // BATCH
            for k in range(SEQ_PER_CHUNK):
                pltpu.async_copy(ob.at[pl.ds(k * BATCH, BATCH)],
                                 out_hbm.at[s0 + k], osems[b])
        return carry

    lax.fori_loop(0, NCHUNK // 2, step, 0)
    wait_store(0)
    wait_store(1)


def _emb_call(widx, pidx, word_table, pos_table):
    mesh = plsc.VectorSubcoreMesh(core_axis_name="c", subcore_axis_name="s")
    f = pl.kernel(
        _emb_body,
        out_type=jax.ShapeDtypeStruct((SEQ, BATCH, HIDDEN), jnp.float32),
        mesh=mesh,
        scratch_types=[
            pltpu.VMEM((ROWS_PER_W,), jnp.int32),
            pltpu.VMEM((ROWS_PER_W,), jnp.int32),
            pltpu.VMEM((CHUNK, HIDDEN), jnp.float32),
            pltpu.VMEM((CHUNK, HIDDEN), jnp.float32),
            pltpu.VMEM((CHUNK, HIDDEN), jnp.float32),
            pltpu.VMEM((CHUNK, HIDDEN), jnp.float32),
            pltpu.VMEM((CHUNK, HIDDEN), jnp.float32),
            pltpu.VMEM((CHUNK, HIDDEN), jnp.float32),
            pltpu.SemaphoreType.DMA,
            pltpu.SemaphoreType.DMA,
            pltpu.SemaphoreType.DMA,
            pltpu.SemaphoreType.DMA,
            pltpu.SemaphoreType.DMA,
            pltpu.SemaphoreType.DMA,
        ],
    )
    return f(widx, pidx, word_table, pos_table)


def kernel(input_ids, position_ids, word_table, pos_table):
    widx = input_ids.T.reshape(-1).astype(jnp.int32)
    pidx = position_ids.T.reshape(-1).astype(jnp.int32)
    return _emb_call(widx, pidx, word_table, pos_table)


# speedup vs baseline: 1.0365x; 1.0365x over previous
import jax
import jax.numpy as jnp
from jax import lax
from jax.experimental import pallas as pl
from jax.experimental.pallas import tpu as pltpu
from jax.experimental.pallas import tpu_sc as plsc

VOCAB = 100000
MAX_POS = 8192
HIDDEN = 1024
BATCH = 4
SEQ = 8192

NUM_ROWS = SEQ * BATCH
NC, NS = 2, 16
NW = NC * NS
ROWS_PER_W = NUM_ROWS // NW
SEQ_PER_W = ROWS_PER_W // BATCH
CHUNK = 16
SEQ_PER_CHUNK = CHUNK // BATCH
NCHUNK = ROWS_PER_W // CHUNK
VREGS_PER_ROW = HIDDEN // 16


def _emb_body(ids_hbm, posids_hbm, word_hbm, pos_hbm, out_hbm,
              idsbuf, widx_v, pidx_v,
              wbuf0, wbuf1, pbuf0, pbuf1, obuf0, obuf1,
              wsem0, wsem1, psem0, psem1, osem0, osem1):
    wbufs = (wbuf0, wbuf1)
    pbufs = (pbuf0, pbuf1)
    obufs = (obuf0, obuf1)
    wsems = (wsem0, wsem1)
    psems = (psem0, psem1)
    osems = (osem0, osem1)

    wid = lax.axis_index("s") * NC + lax.axis_index("c")
    base = wid * ROWS_PER_W
    s_base = wid * SEQ_PER_W

    def load_indices(src_hbm, dst_v):
        for b in range(BATCH):
            pltpu.sync_copy(src_hbm.at[b, pl.ds(s_base, SEQ_PER_W)],
                            idsbuf.at[b])

        def interleave(k, carry):
            t = k * 16 + lax.iota(jnp.int32, 16)
            row = jnp.bitwise_and(t, BATCH - 1)
            col = lax.shift_right_logical(t, 2)
            dst_v[pl.ds(k * 16, 16)] = plsc.load_gather(idsbuf, [row, col])
            return carry

        lax.fori_loop(0, ROWS_PER_W // 16, interleave, 0)

    load_indices(ids_hbm, widx_v)
    load_indices(posids_hbm, pidx_v)

    def issue_gathers(c, b):
        off = c * CHUNK
        pltpu.async_copy(word_hbm.at[widx_v.at[pl.ds(off, CHUNK)]],
                         wbufs[b], wsems[b])
        pltpu.async_copy(pos_hbm.at[pidx_v.at[pl.ds(off, CHUNK)]],
                         pbufs[b], psems[b])

    def wait_gathers(b):
        pltpu.make_async_copy(word_hbm.at[pl.ds(0, CHUNK)],
                              wbufs[b], wsems[b]).wait()
        pltpu.make_async_copy(pos_hbm.at[pl.ds(0, CHUNK)],
                              pbufs[b], psems[b]).wait()

    def wait_store(b):
        for _ in range(SEQ_PER_CHUNK):
            pltpu.make_async_copy(obufs[b].at[pl.ds(0, BATCH)],
                                  out_hbm.at[0], osems[b]).wait()

    issue_gathers(0, 0)
    issue_gathers(1, 1)

    def step(i, carry):
        for b in range(2):
            c = i * 2 + b
            wait_gathers(b)
            @pl.when(i >= 1)
            def _():
                wait_store(b)
            wb, pb, ob = wbufs[b], pbufs[b], obufs[b]

            def add_row(r, carry2):
                for v in range(VREGS_PER_ROW):
                    sl = pl.ds(v * 16, 16)
                    ob[r, sl] = wb[r, sl] + pb[r, sl]
                return carry2

            lax.fori_loop(0, CHUNK, add_row, 0)

            @pl.when(i < (NCHUNK // 2) - 1)
            def _():
                issue_gathers(c + 2, b)
            s0 = (base + c * CHUNK) // BATCH
            for k in range(SEQ_PER_CHUNK):
                pltpu.async_copy(ob.at[pl.ds(k * BATCH, BATCH)],
                                 out_hbm.at[s0 + k], osems[b])
        return carry

    lax.fori_loop(0, NCHUNK // 2, step, 0)
    wait_store(0)
    wait_store(1)


def _emb_call(input_ids, position_ids, word_table, pos_table):
    mesh = plsc.VectorSubcoreMesh(core_axis_name="c", subcore_axis_name="s")
    f = pl.kernel(
        _emb_body,
        out_type=jax.ShapeDtypeStruct((SEQ, BATCH, HIDDEN), jnp.float32),
        mesh=mesh,
        scratch_types=[
            pltpu.VMEM((BATCH, SEQ_PER_W), jnp.int32),
            pltpu.VMEM((ROWS_PER_W,), jnp.int32),
            pltpu.VMEM((ROWS_PER_W,), jnp.int32),
            pltpu.VMEM((CHUNK, HIDDEN), jnp.float32),
            pltpu.VMEM((CHUNK, HIDDEN), jnp.float32),
            pltpu.VMEM((CHUNK, HIDDEN), jnp.float32),
            pltpu.VMEM((CHUNK, HIDDEN), jnp.float32),
            pltpu.VMEM((CHUNK, HIDDEN), jnp.float32),
            pltpu.VMEM((CHUNK, HIDDEN), jnp.float32),
            pltpu.SemaphoreType.DMA,
            pltpu.SemaphoreType.DMA,
            pltpu.SemaphoreType.DMA,
            pltpu.SemaphoreType.DMA,
            pltpu.SemaphoreType.DMA,
            pltpu.SemaphoreType.DMA,
        ],
        compiler_params=pltpu.CompilerParams(needs_layout_passes=False),
    )
    return f(input_ids, position_ids, word_table, pos_table)


def kernel(input_ids, position_ids, word_table, pos_table):
    return _emb_call(input_ids.astype(jnp.int32),
                     position_ids.astype(jnp.int32),
                     word_table, pos_table)


# speedup vs baseline: 1.0415x; 1.0049x over previous
import jax
import jax.numpy as jnp
from jax import lax
from jax.experimental import pallas as pl
from jax.experimental.pallas import tpu as pltpu
from jax.experimental.pallas import tpu_sc as plsc

VOCAB = 100000
MAX_POS = 8192
HIDDEN = 1024
BATCH = 4
SEQ = 8192

NUM_ROWS = SEQ * BATCH
NC, NS = 2, 16
NW = NC * NS
ROWS_PER_W = NUM_ROWS // NW
SEQ_PER_W = ROWS_PER_W // BATCH
CHUNK = 8
SEQ_PER_CHUNK = CHUNK // BATCH
NCHUNK = ROWS_PER_W // CHUNK
NSLOT = 4
VREGS_PER_ROW = HIDDEN // 16


def _emb_body(ids_hbm, posids_hbm, word_hbm, pos_hbm, out_hbm,
              idsbuf, widx_v, pidx_v, *rest):
    wbufs = rest[0:NSLOT]
    pbufs = rest[NSLOT:2 * NSLOT]
    obufs = rest[2 * NSLOT:3 * NSLOT]
    wsems = rest[3 * NSLOT:4 * NSLOT]
    psems = rest[4 * NSLOT:5 * NSLOT]
    osems = rest[5 * NSLOT:6 * NSLOT]

    wid = lax.axis_index("s") * NC + lax.axis_index("c")
    base = wid * ROWS_PER_W
    s_base = wid * SEQ_PER_W

    def load_indices(src_hbm, dst_v):
        for b in range(BATCH):
            pltpu.sync_copy(src_hbm.at[b, pl.ds(s_base, SEQ_PER_W)],
                            idsbuf.at[b])

        def interleave(k, carry):
            t = k * 16 + lax.iota(jnp.int32, 16)
            row = jnp.bitwise_and(t, BATCH - 1)
            col = lax.shift_right_logical(t, 2)
            dst_v[pl.ds(k * 16, 16)] = plsc.load_gather(idsbuf, [row, col])
            return carry

        lax.fori_loop(0, ROWS_PER_W // 16, interleave, 0)

    load_indices(ids_hbm, widx_v)
    load_indices(posids_hbm, pidx_v)

    def issue_gathers(c, b):
        off = c * CHUNK
        pltpu.async_copy(word_hbm.at[widx_v.at[pl.ds(off, CHUNK)]],
                         wbufs[b], wsems[b])
        pltpu.async_copy(pos_hbm.at[pidx_v.at[pl.ds(off, CHUNK)]],
                         pbufs[b], psems[b])

    def wait_gathers(b):
        pltpu.make_async_copy(word_hbm.at[pl.ds(0, CHUNK)],
                              wbufs[b], wsems[b]).wait()
        pltpu.make_async_copy(pos_hbm.at[pl.ds(0, CHUNK)],
                              pbufs[b], psems[b]).wait()

    def wait_store(b):
        for _ in range(SEQ_PER_CHUNK):
            pltpu.make_async_copy(obufs[b].at[pl.ds(0, BATCH)],
                                  out_hbm.at[0], osems[b]).wait()

    for b in range(NSLOT):
        issue_gathers(b, b)

    def step(i, carry):
        for b in range(NSLOT):
            c = i * NSLOT + b
            wait_gathers(b)
            @pl.when(i >= 1)
            def _():
                wait_store(b)
            wb, pb, ob = wbufs[b], pbufs[b], obufs[b]

            def add_row(r, carry2):
                for v in range(VREGS_PER_ROW):
                    sl = pl.ds(v * 16, 16)
                    ob[r, sl] = wb[r, sl] + pb[r, sl]
                return carry2

            lax.fori_loop(0, CHUNK, add_row, 0)

            @pl.when(i < (NCHUNK // NSLOT) - 1)
            def _():
                issue_gathers(c + NSLOT, b)
            s0 = (base + c * CHUNK) // BATCH
            for k in range(SEQ_PER_CHUNK):
                pltpu.async_copy(ob.at[pl.ds(k * BATCH, BATCH)],
                                 out_hbm.at[s0 + k], osems[b])
        return carry

    lax.fori_loop(0, NCHUNK // NSLOT, step, 0)
    for b in range(NSLOT):
        wait_store(b)


def _emb_call(input_ids, position_ids, word_table, pos_table):
    mesh = plsc.VectorSubcoreMesh(core_axis_name="c", subcore_axis_name="s")
    f = pl.kernel(
        _emb_body,
        out_type=jax.ShapeDtypeStruct((SEQ, BATCH, HIDDEN), jnp.float32),
        mesh=mesh,
        scratch_types=(
            [pltpu.VMEM((BATCH, SEQ_PER_W), jnp.int32),
             pltpu.VMEM((ROWS_PER_W,), jnp.int32),
             pltpu.VMEM((ROWS_PER_W,), jnp.int32)]
            + [pltpu.VMEM((CHUNK, HIDDEN), jnp.float32)] * (3 * NSLOT)
            + [pltpu.SemaphoreType.DMA] * (3 * NSLOT)
        ),
        compiler_params=pltpu.CompilerParams(needs_layout_passes=False),
    )
    return f(input_ids, position_ids, word_table, pos_table)


def kernel(input_ids, position_ids, word_table, pos_table):
    return _emb_call(input_ids.astype(jnp.int32),
                     position_ids.astype(jnp.int32),
                     word_table, pos_table)
